# baseline (speedup 1.0000x reference)
import functools

import jax
import jax.numpy as jnp
from jax import lax
from jax.experimental import pallas as pl
from jax.experimental.pallas import tpu as pltpu
from jax.experimental.pallas import tpu_sc as plsc

_NEG = -1e37


def _sc_gather(W, b, sample_ids, labels):
    V, D = W.shape
    S = sample_ids.shape[0]
    B = labels.shape[0]
    info = plsc.get_sparse_core_info()
    NC, NS = info.num_cores, info.num_subcores
    NW = NC * NS
    s_per = S // NW
    l_per = B // NW
    CH = 128
    s_ch = s_per // CH
    l_ch = l_per // CH

    sid3 = sample_ids.reshape(NW, s_ch, CH)
    lab3 = labels.reshape(NW, l_ch, CH)

    mesh = plsc.VectorSubcoreMesh(core_axis_name="c", subcore_axis_name="s")

    @functools.partial(
        pl.kernel,
        mesh=mesh,
        compiler_params=pltpu.CompilerParams(use_tc_tiling_on_sc=False),
        out_type=[
            jax.ShapeDtypeStruct((S, D), jnp.float32),
            jax.ShapeDtypeStruct((S,), jnp.float32),
            jax.ShapeDtypeStruct((B, D), jnp.float32),
            jax.ShapeDtypeStruct((B,), jnp.float32),
        ],
        scratch_types=[
            pltpu.VMEM((s_ch, CH), jnp.int32),
            pltpu.VMEM((s_per, D), jnp.float32),
            pltpu.VMEM((s_per,), jnp.float32),
            pltpu.VMEM((l_ch, CH), jnp.int32),
            pltpu.VMEM((l_per, D), jnp.float32),
            pltpu.VMEM((l_per,), jnp.float32),
            pltpu.SemaphoreType.DMA,
        ],
    )
    def gather_k(w_hbm, b_hbm, sid_hbm, lab_hbm,
                 sw_out, sb_out, tw_out, tb_out,
                 sidx_v, srows_v, sb_v, lidx_v, lrows_v, lb_v, sem):
        wid = lax.axis_index("s") * NC + lax.axis_index("c")
        sbase = wid * s_per
        lbase = wid * l_per
        pltpu.sync_copy(sid_hbm.at[wid], sidx_v)
        pltpu.sync_copy(lab_hbm.at[wid], lidx_v)
        handles = []
        for j in range(s_ch):
            handles.append(pltpu.async_copy(
                w_hbm.at[sidx_v.at[j]],
                srows_v.at[pl.ds(j * CH, CH), :], sem))
            handles.append(pltpu.async_copy(
                b_hbm.at[sidx_v.at[j]],
                sb_v.at[pl.ds(j * CH, CH)], sem))
        for j in range(l_ch):
            handles.append(pltpu.async_copy(
                w_hbm.at[lidx_v.at[j]],
                lrows_v.at[pl.ds(j * CH, CH), :], sem))
            handles.append(pltpu.async_copy(
                b_hbm.at[lidx_v.at[j]],
                lb_v.at[pl.ds(j * CH, CH)], sem))
        for h in handles:
            h.wait()
        pltpu.sync_copy(srows_v, sw_out.at[pl.ds(sbase, s_per)])
        pltpu.sync_copy(sb_v, sb_out.at[pl.ds(sbase, s_per)])
        pltpu.sync_copy(lrows_v, tw_out.at[pl.ds(lbase, l_per)])
        pltpu.sync_copy(lb_v, tb_out.at[pl.ds(lbase, l_per)])

    return gather_k(W, b, sid3, lab3)


def _tc_body(x_ref, wt_ref, ids_ref, bias_ref, sfreq_ref,
             lab_ref, tw_ref, tb_ref, tf_ref, out_ref):
    x = x_ref[...]
    s = lax.dot_general(x, wt_ref[...], (((1,), (0,)), ((), ())),
                        preferred_element_type=jnp.float32)
    row = bias_ref[...] - jnp.log(sfreq_ref[...])
    s = s + row
    hit = lab_ref[...] == ids_ref[...]
    s = jnp.where(hit, jnp.float32(_NEG), s)
    t = (jnp.sum(x * tw_ref[...], axis=1, keepdims=True)
         + tb_ref[...] - jnp.log(tf_ref[...]))
    out_ref[...] = s
    out_ref[:, 0:1] = t


def _tc_logits(x, wt_p, ids_p, bias_p, sfreq_p, lab2, tw, tb2, tf2):
    BATCH, D = x.shape
    W1 = wt_p.shape[1]
    BR = 256
    nb = BATCH // BR
    return pl.pallas_call(
        _tc_body,
        grid=(nb,),
        in_specs=[
            pl.BlockSpec((BR, D), lambda i: (i, 0)),
            pl.BlockSpec((D, W1), lambda i: (0, 0)),
            pl.BlockSpec((1, W1), lambda i: (0, 0)),
            pl.BlockSpec((1, W1), lambda i: (0, 0)),
            pl.BlockSpec((1, W1), lambda i: (0, 0)),
            pl.BlockSpec((BR, 1), lambda i: (i, 0)),
            pl.BlockSpec((BR, D), lambda i: (i, 0)),
            pl.BlockSpec((BR, 1), lambda i: (i, 0)),
            pl.BlockSpec((BR, 1), lambda i: (i, 0)),
        ],
        out_specs=pl.BlockSpec((BR, W1), lambda i: (i, 0)),
        out_shape=jax.ShapeDtypeStruct((BATCH, W1), jnp.float32),
        compiler_params=pltpu.CompilerParams(
            dimension_semantics=("arbitrary",)),
    )(x, wt_p, ids_p, bias_p, sfreq_p, lab2, tw, tb2, tf2)


def kernel(inputs, W, b, true_freq, sample_freq, labels, sample_ids):
    sw, sb, tw, tb = _sc_gather(W, b, sample_ids, labels)

    zc = jnp.zeros((W.shape[1], 1), jnp.float32)
    wt_p = jnp.concatenate([zc, sw.T], axis=1)
    ids_p = jnp.concatenate(
        [jnp.full((1,), -1, jnp.int32), sample_ids])[None, :]
    bias_p = jnp.concatenate(
        [jnp.zeros((1,), jnp.float32), sb])[None, :]
    sfreq_p = jnp.concatenate(
        [jnp.ones((1,), jnp.float32), sample_freq])[None, :]

    return _tc_logits(inputs, wt_p, ids_p, bias_p, sfreq_p,
                      labels[:, None], tw, tb[:, None], true_freq[:, None])
